# baseline (speedup 1.0000x reference)
import functools

import jax
import jax.numpy as jnp
from jax import lax
from jax.experimental import pallas as pl
from jax.experimental.pallas import tpu as pltpu
from jax.experimental.pallas import tpu_sc as plsc

_THR = 0.99


def _sc_scan_body(T, B, CH, w_hbm, pad_hbm, c_hbm, r_hbm, q_hbm, qe_hbm,
                  w_v, c_v, r_v, q_v, qe_v, pad_v):
    cid = lax.axis_index("c")
    sid = lax.axis_index("s")

    @pl.when(jnp.logical_and(cid == 0, sid == 0))
    def _():
        pltpu.sync_copy(pad_hbm, pad_v)
        pad = pad_v[...]
        thr = jnp.full((B,), _THR, jnp.float32)
        one = jnp.full((B,), 1.0, jnp.float32)
        zero = jnp.zeros((B,), jnp.float32)
        prev0 = jnp.zeros((B,), jnp.float32)
        qc0 = jnp.zeros((B,), jnp.float32)
        prev, qc = prev0, qc0
        for ch in range(T // CH):
            pltpu.sync_copy(w_hbm.at[pl.ds(ch * CH * B, CH * B)], w_v)

            def body(t, carry):
                prev, qc = carry
                w = w_v[pl.ds(t * B, B)]
                s = prev + w
                fired = s >= thr
                rem = one - prev
                wmr = w - rem
                prev_n = jnp.where(fired, wmr, s)
                c_v[pl.ds(t * B, B)] = jnp.where(fired, wmr, w)
                tf = jnp.full((B,), ch * CH, jnp.float32) + lax.convert_element_type(
                    jnp.full((B,), 1, jnp.int32) * t, jnp.float32)
                o = jnp.logical_and(fired, pad >= tf)
                r_v[pl.ds(t * B, B)] = jnp.where(o, rem, zero)
                q_v[pl.ds(t * B, B)] = qc
                qc_n = qc + jnp.where(fired, one, zero)
                qe_v[pl.ds(t * B, B)] = qc_n
                return prev_n, qc_n

            prev, qc = lax.fori_loop(0, CH, body, (prev, qc))
            pltpu.sync_copy(c_v, c_hbm.at[pl.ds(ch * CH * B, CH * B)])
            pltpu.sync_copy(r_v, r_hbm.at[pl.ds(ch * CH * B, CH * B)])
            pltpu.sync_copy(q_v, q_hbm.at[pl.ds(ch * CH * B, CH * B)])
            pltpu.sync_copy(qe_v, qe_hbm.at[pl.ds(ch * CH * B, CH * B)])


def _sc_scan(wT, pad_start):
    T, B = wT.shape
    CH = 1024
    body = functools.partial(_sc_scan_body, T, B, CH)
    call = pl.kernel(
        body,
        out_type=[jax.ShapeDtypeStruct((T * B,), jnp.float32)
                  for _ in range(4)],
        mesh=plsc.VectorSubcoreMesh(core_axis_name="c", subcore_axis_name="s"),
        scratch_types=[pltpu.VMEM((CH * B,), jnp.float32) for _ in range(5)]
        + [pltpu.VMEM((B,), jnp.float32)],
    )
    c, r2, q, qe = call(wT.reshape(-1), pad_start)
    return (c.reshape(T, B), r2.reshape(T, B), q.reshape(T, B),
            qe.reshape(T, B))


def _combine_body(S, B, nT, x_hbm, c_ref, r_ref, q_ref, qe_ref, out_ref,
                  xbuf, sem, carry_ref):
    g = pl.program_id(0)
    it = g // B
    b = g % B

    def start(ci):
        pltpu.make_async_copy(
            x_hbm.at[pl.ds(ci * S, S)],
            xbuf.at[ci % 3],
            sem.at[ci % 3],
        ).start()

    @pl.when(g == 0)
    def _():
        start(0)
        start(1)
        carry_ref[...] = jnp.zeros_like(carry_ref)

    @pl.when(jnp.logical_and(b == 0, it + 2 < nT))
    def _():
        start(it + 2)

    @pl.when(b == 0)
    def _():
        pltpu.make_async_copy(
            x_hbm.at[pl.ds(it * S, S)],
            xbuf.at[it % 3], sem.at[it % 3]).wait()

    x = xbuf[it % 3, :, b, :]

    lane = lax.broadcasted_iota(jnp.int32, (1, B), 1)
    eb = (lane == b).astype(jnp.float32)
    cc = jnp.sum(c_ref[...] * eb, axis=1, keepdims=True)
    rr = jnp.sum(r_ref[...] * eb, axis=1, keepdims=True)
    qq = jnp.sum(q_ref[...] * eb, axis=1, keepdims=True)
    qe = qe_ref[0]

    ti = lax.broadcasted_iota(jnp.int32, (S, 1), 0)
    ki = lax.broadcasted_iota(jnp.int32, (1, S), 1)
    M = jnp.logical_and(ki < ti, qq == qe).astype(jnp.float32)

    y = cc * x
    ps = jnp.dot(M, y, preferred_element_type=jnp.float32)

    q0 = qq[0:1, 0:1]
    gm = (qq == q0).astype(jnp.float32)
    carry_b = carry_ref[pl.ds(b, 1), :]
    ps = ps + gm * carry_b
    o = (rr > 0).astype(jnp.float32)
    out_ref[0] = o * ps + rr * x

    qel = qe[0:1, S - 1:S]
    mrow = (qe == qel).astype(jnp.float32)
    newc = jnp.dot(mrow, y, preferred_element_type=jnp.float32)

    nof = (qel == q0).astype(jnp.float32)
    carry_ref[pl.ds(b, 1), :] = newc + nof * carry_b


def _combine(x, c, r2, q, qe3, S):
    T, B, C = x.shape
    nT = T // S
    body = functools.partial(_combine_body, S, B, nT)
    return pl.pallas_call(
        body,
        grid=(B * nT,),
        in_specs=[
            pl.BlockSpec(memory_space=pl.ANY),
            pl.BlockSpec((S, B), lambda g: (g // B, 0)),
            pl.BlockSpec((S, B), lambda g: (g // B, 0)),
            pl.BlockSpec((S, B), lambda g: (g // B, 0)),
            pl.BlockSpec((1, 1, S), lambda g: ((g % B) * nT + g // B, 0, 0)),
        ],
        out_specs=pl.BlockSpec((1, S, C), lambda g: (g % B, g // B, 0)),
        out_shape=jax.ShapeDtypeStruct((B, T, C), jnp.float32),
        scratch_shapes=[pltpu.VMEM((3, S, B, C), jnp.float32),
                        pltpu.SemaphoreType.DMA((3,)),
                        pltpu.VMEM((B, C), jnp.float32)],
        compiler_params=pltpu.CompilerParams(
            dimension_semantics=("arbitrary",)),
    )(x, c, r2, q, qe3)


def kernel(encoder_out, encoder_padding_mask, w_proj, b_proj):
    x = jnp.transpose(encoder_out, (1, 0, 2))
    B, T, C = x.shape
    sig = jnp.einsum('btc,c->bt', x, w_proj) + b_proj
    weight = jax.nn.sigmoid(sig)
    not_pad = ~encoder_padding_mask
    weight = weight * not_pad.astype(weight.dtype)
    pad_start = not_pad.sum(-1).astype(jnp.float32)

    wT = weight.T
    c, r2, q, qe = _sc_scan(wT, pad_start)

    S = 128
    nT = T // S
    qe3 = qe.T.reshape(B * nT, 1, S)
    return _combine(encoder_out, c, r2, q, qe3, S)


# speedup vs baseline: 1.5801x; 1.5801x over previous
import functools

import jax
import jax.numpy as jnp
from jax import lax
from jax.experimental import pallas as pl
from jax.experimental.pallas import tpu as pltpu
from jax.experimental.pallas import tpu_sc as plsc

_THR = 0.99


def _sc_scan_body(T, B, CH, w_hbm, pad_hbm, c_hbm, r_hbm, q_hbm, qe_hbm,
                  w_v, c_v, r_v, q_v, qe_v, pad_v):
    cid = lax.axis_index("c")
    sid = lax.axis_index("s")

    @pl.when(jnp.logical_and(cid == 0, sid == 0))
    def _():
        pltpu.sync_copy(pad_hbm, pad_v)
        pad = pad_v[...]
        thr = jnp.full((B,), _THR, jnp.float32)
        one = jnp.full((B,), 1.0, jnp.float32)
        zero = jnp.zeros((B,), jnp.float32)
        prev0 = jnp.zeros((B,), jnp.float32)
        qc0 = jnp.zeros((B,), jnp.float32)
        prev, qc = prev0, qc0
        for ch in range(T // CH):
            pltpu.sync_copy(w_hbm.at[pl.ds(ch * CH * B, CH * B)], w_v)

            def body(t, carry):
                prev, qc = carry
                w = w_v[pl.ds(t * B, B)]
                s = prev + w
                fired = s >= thr
                rem = one - prev
                wmr = w - rem
                prev_n = jnp.where(fired, wmr, s)
                c_v[pl.ds(t * B, B)] = jnp.where(fired, wmr, w)
                tf = jnp.full((B,), ch * CH, jnp.float32) + lax.convert_element_type(
                    jnp.full((B,), 1, jnp.int32) * t, jnp.float32)
                o = jnp.logical_and(fired, pad >= tf)
                r_v[pl.ds(t * B, B)] = jnp.where(o, rem, zero)
                q_v[pl.ds(t * B, B)] = qc
                qc_n = qc + jnp.where(fired, one, zero)
                qe_v[pl.ds(t * B, B)] = qc_n
                return prev_n, qc_n

            prev, qc = lax.fori_loop(0, CH, body, (prev, qc))
            pltpu.sync_copy(c_v, c_hbm.at[pl.ds(ch * CH * B, CH * B)])
            pltpu.sync_copy(r_v, r_hbm.at[pl.ds(ch * CH * B, CH * B)])
            pltpu.sync_copy(q_v, q_hbm.at[pl.ds(ch * CH * B, CH * B)])
            pltpu.sync_copy(qe_v, qe_hbm.at[pl.ds(ch * CH * B, CH * B)])


def _sc_scan(wT, pad_start):
    T, B = wT.shape
    CH = 1024
    body = functools.partial(_sc_scan_body, T, B, CH)
    call = pl.kernel(
        body,
        out_type=[jax.ShapeDtypeStruct((T * B,), jnp.float32)
                  for _ in range(4)],
        mesh=plsc.VectorSubcoreMesh(core_axis_name="c", subcore_axis_name="s"),
        scratch_types=[pltpu.VMEM((CH * B,), jnp.float32) for _ in range(5)]
        + [pltpu.VMEM((B,), jnp.float32)],
    )
    c, r2, q, qe = call(wT.reshape(-1), pad_start)
    return (c.reshape(T, B), r2.reshape(T, B), q.reshape(T, B),
            qe.reshape(T, B))


def _combine_body(S, B, nT, x_hbm, c_ref, r_ref, q_ref, qe_ref, out_ref,
                  xbuf, sem, carry_ref):
    it = pl.program_id(0)

    def start(ci):
        pltpu.make_async_copy(
            x_hbm.at[pl.ds(ci * S, S)],
            xbuf.at[ci % 3],
            sem.at[ci % 3],
        ).start()

    @pl.when(it == 0)
    def _():
        start(0)
        start(1)
        carry_ref[...] = jnp.zeros_like(carry_ref)

    @pl.when(it + 2 < nT)
    def _():
        start(it + 2)

    pltpu.make_async_copy(
        x_hbm.at[pl.ds(it * S, S)],
        xbuf.at[it % 3], sem.at[it % 3]).wait()

    slot = it % 3
    cb_all = c_ref[...]
    rb_all = r_ref[...]
    qb_all = q_ref[...]
    lane = lax.broadcasted_iota(jnp.int32, (1, B), 1)
    ti = lax.broadcasted_iota(jnp.int32, (S, 1), 0)
    ki = lax.broadcasted_iota(jnp.int32, (1, S), 1)
    tri = ki < ti

    for b in range(B):
        x = xbuf[slot, :, b, :]
        eb = (lane == b).astype(jnp.float32)
        cc = jnp.sum(cb_all * eb, axis=1, keepdims=True)
        rr = jnp.sum(rb_all * eb, axis=1, keepdims=True)
        qq = jnp.sum(qb_all * eb, axis=1, keepdims=True)
        qe = qe_ref[0, b:b + 1, :]

        M = jnp.logical_and(tri, qq == qe).astype(jnp.float32)
        y = cc * x
        ps = jnp.dot(M, y, preferred_element_type=jnp.float32)

        q0 = qq[0:1, 0:1]
        gm = (qq == q0).astype(jnp.float32)
        carry_b = carry_ref[b:b + 1, :]
        ps = ps + gm * carry_b
        o = (rr > 0).astype(jnp.float32)
        out_ref[b] = o * ps + rr * x

        qel = qe[0:1, S - 1:S]
        mrow = (qe == qel).astype(jnp.float32)
        newc = jnp.dot(mrow, y, preferred_element_type=jnp.float32)
        nof = (qel == q0).astype(jnp.float32)
        carry_ref[b:b + 1, :] = newc + nof * carry_b


def _combine(x, c, r2, q, qe3, S):
    T, B, C = x.shape
    nT = T // S
    body = functools.partial(_combine_body, S, B, nT)
    return pl.pallas_call(
        body,
        grid=(nT,),
        in_specs=[
            pl.BlockSpec(memory_space=pl.ANY),
            pl.BlockSpec((S, B), lambda g: (g, 0)),
            pl.BlockSpec((S, B), lambda g: (g, 0)),
            pl.BlockSpec((S, B), lambda g: (g, 0)),
            pl.BlockSpec((1, B, S), lambda g: (g, 0, 0)),
        ],
        out_specs=pl.BlockSpec((B, S, C), lambda g: (0, g, 0)),
        out_shape=jax.ShapeDtypeStruct((B, T, C), jnp.float32),
        scratch_shapes=[pltpu.VMEM((3, S, B, C), jnp.float32),
                        pltpu.SemaphoreType.DMA((3,)),
                        pltpu.VMEM((B, C), jnp.float32)],
        compiler_params=pltpu.CompilerParams(
            dimension_semantics=("arbitrary",)),
    )(x, c, r2, q, qe3)


def kernel(encoder_out, encoder_padding_mask, w_proj, b_proj):
    x = jnp.transpose(encoder_out, (1, 0, 2))
    B, T, C = x.shape
    sig = jnp.einsum('btc,c->bt', x, w_proj) + b_proj
    weight = jax.nn.sigmoid(sig)
    not_pad = ~encoder_padding_mask
    weight = weight * not_pad.astype(weight.dtype)
    pad_start = not_pad.sum(-1).astype(jnp.float32)

    wT = weight.T
    c, r2, q, qe = _sc_scan(wT, pad_start)

    S = 128
    nT = T // S
    qe3 = qe.T.reshape(B, nT, S).transpose(1, 0, 2)
    return _combine(encoder_out, c, r2, q, qe3, S)
